# baseline (speedup 1.0000x reference)
import jax
import jax.numpy as jnp
from jax.experimental import pallas as pl
from jax.experimental.pallas import tpu as pltpu

_FAST_CHUNKS = 16


def _pack_body(idx_ref, in_ref, fast_ref, slow_ref, copy_sems, gather_sems):
    T = in_ref.shape[1]
    num = slow_ref.shape[1]
    span = T // _FAST_CHUNKS
    copies = []
    for k in range(_FAST_CHUNKS):
        copies.append(
            pltpu.make_async_copy(
                in_ref.at[:, k * span : (k + 1) * span],
                fast_ref.at[:, k * span : (k + 1) * span],
                copy_sems.at[k],
            )
        )
    for p in range(num):
        copies.append(
            pltpu.make_async_copy(
                in_ref.at[:, pl.ds(idx_ref[p], 1)],
                slow_ref.at[:, pl.ds(p, 1)],
                gather_sems.at[p],
            )
        )
    for c in copies:
        c.start()
    for c in copies:
        c.wait()


def kernel(frames, slowfast_alpha):
    del slowfast_alpha
    C, T, H, W = frames.shape
    num = T // 4
    idx = jnp.linspace(0.0, T - 1, num).astype(jnp.int32)

    grid_spec = pltpu.PrefetchScalarGridSpec(
        num_scalar_prefetch=1,
        grid=(1,),
        in_specs=[pl.BlockSpec(memory_space=pltpu.MemorySpace.HBM)],
        out_specs=[
            pl.BlockSpec(memory_space=pltpu.MemorySpace.HBM),
            pl.BlockSpec(memory_space=pltpu.MemorySpace.HBM),
        ],
        scratch_shapes=[
            pltpu.SemaphoreType.DMA((_FAST_CHUNKS,)),
            pltpu.SemaphoreType.DMA((num,)),
        ],
    )
    fast, slow = pl.pallas_call(
        _pack_body,
        grid_spec=grid_spec,
        out_shape=[
            jax.ShapeDtypeStruct((C, T, H, W), frames.dtype),
            jax.ShapeDtypeStruct((C, num, H, W), frames.dtype),
        ],
    )(idx, frames)
    return (slow, fast)


# speedup vs baseline: 24.4017x; 24.4017x over previous
import functools

import jax
import jax.numpy as jnp
from jax import lax
from jax.experimental import pallas as pl
from jax.experimental.pallas import tpu as pltpu
from jax.experimental.pallas import tpu_sc as plsc

_CHUNK = 128


def _copy_body(in_ref, out_ref):
    out_ref[...] = in_ref[...]


def _tc_copy(frames):
    C, T, H, W = frames.shape
    return pl.pallas_call(
        _copy_body,
        grid=(T,),
        in_specs=[pl.BlockSpec((C, 1, H, W), lambda t: (0, t, 0, 0))],
        out_specs=pl.BlockSpec((C, 1, H, W), lambda t: (0, t, 0, 0)),
        out_shape=jax.ShapeDtypeStruct((C, T, H, W), frames.dtype),
    )(frames)


def _sc_gather(table, idx_arr, n_out_rows, n_chunks):
    W = table.shape[-1]
    mesh = plsc.VectorSubcoreMesh(core_axis_name="c", subcore_axis_name="s")
    num_cores = mesh.num_cores

    @functools.partial(
        pl.kernel,
        out_type=jax.ShapeDtypeStruct((n_out_rows, W), table.dtype),
        mesh=mesh,
        scratch_types=[
            pltpu.VMEM((n_chunks, _CHUNK), jnp.int32),
            pltpu.VMEM((n_chunks, _CHUNK, W), table.dtype),
            pltpu.SemaphoreType.DMA,
        ],
    )
    def gather_kernel(table_hbm, idx_hbm, out_hbm, idx_v, rows_v, sem):
        wid = lax.axis_index("s") * num_cores + lax.axis_index("c")
        pltpu.sync_copy(idx_hbm.at[wid], idx_v)
        gathers = [
            pltpu.async_copy(table_hbm.at[idx_v.at[j]], rows_v.at[j], sem)
            for j in range(n_chunks)
        ]
        for g in gathers:
            g.wait()
        base = wid * (n_chunks * _CHUNK)
        writes = [
            pltpu.async_copy(
                rows_v.at[j], out_hbm.at[pl.ds(base + j * _CHUNK, _CHUNK)], sem
            )
            for j in range(n_chunks)
        ]
        for w in writes:
            w.wait()

    return gather_kernel(table, idx_arr)


def kernel(frames, slowfast_alpha):
    del slowfast_alpha
    C, T, H, W = frames.shape
    num = T // 4
    idx = jnp.linspace(0.0, T - 1, num).astype(jnp.int32)

    frame_rows = (jnp.arange(C, dtype=jnp.int32)[:, None] * T + idx[None, :]).reshape(-1)
    row_idx = (
        frame_rows[:, None] * H + jnp.arange(H, dtype=jnp.int32)[None, :]
    ).reshape(-1)
    n_out_rows = C * num * H
    info = plsc.get_sparse_core_info()
    n_workers = info.num_cores * info.num_subcores
    n_chunks = n_out_rows // (n_workers * _CHUNK)
    idx_arr = row_idx.reshape(n_workers, n_chunks, _CHUNK)

    table = frames.reshape(C * T * H, W)
    slow = _sc_gather(table, idx_arr, n_out_rows, n_chunks).reshape(C, num, H, W)
    fast = _tc_copy(frames)
    return (slow, fast)


# speedup vs baseline: 45.6570x; 1.8711x over previous
import jax
import jax.numpy as jnp
from jax.experimental import pallas as pl
from jax.experimental.pallas import tpu as pltpu

_NBUF = 16
_LAG = 8


def _make_schedule(T, num):
    sched = []
    p = 0
    for t in range(T):
        sched.append(("fast", t))
        if t % 4 == 3 and p < num:
            sched.append(("slow", p))
            p += 1
    while p < num:
        sched.append(("slow", p))
        p += 1
    return sched


def _pack_body(idx_ref, in_ref, fast_ref, slow_ref, buf, in_sems, out_sems):
    T = in_ref.shape[1]
    num = slow_ref.shape[1]
    sched = _make_schedule(T, num)
    n = len(sched)
    in_descs = [None] * n
    out_descs = [None] * n
    for i in range(n + _LAG):
        if i < n:
            b = i % _NBUF
            if i >= _NBUF:
                out_descs[i - _NBUF].wait()
            kind, k = sched[i]
            if kind == "fast":
                src = in_ref.at[:, pl.ds(k, 1)]
            else:
                src = in_ref.at[:, pl.ds(idx_ref[k], 1)]
            in_descs[i] = pltpu.make_async_copy(src, buf.at[b], in_sems.at[b])
            in_descs[i].start()
        j = i - _LAG
        if 0 <= j < n:
            bj = j % _NBUF
            in_descs[j].wait()
            kind, k = sched[j]
            if kind == "fast":
                dst = fast_ref.at[:, pl.ds(k, 1)]
            else:
                dst = slow_ref.at[:, pl.ds(k, 1)]
            out_descs[j] = pltpu.make_async_copy(buf.at[bj], dst, out_sems.at[bj])
            out_descs[j].start()
    for j in range(n - _NBUF, n):
        out_descs[j].wait()


def kernel(frames, slowfast_alpha):
    del slowfast_alpha
    C, T, H, W = frames.shape
    num = T // 4
    idx = jnp.linspace(0.0, T - 1, num).astype(jnp.int32)

    grid_spec = pltpu.PrefetchScalarGridSpec(
        num_scalar_prefetch=1,
        grid=(1,),
        in_specs=[pl.BlockSpec(memory_space=pltpu.MemorySpace.HBM)],
        out_specs=[
            pl.BlockSpec(memory_space=pltpu.MemorySpace.HBM),
            pl.BlockSpec(memory_space=pltpu.MemorySpace.HBM),
        ],
        scratch_shapes=[
            pltpu.VMEM((_NBUF, C, 1, H, W), frames.dtype),
            pltpu.SemaphoreType.DMA((_NBUF,)),
            pltpu.SemaphoreType.DMA((_NBUF,)),
        ],
    )
    fast, slow = pl.pallas_call(
        _pack_body,
        grid_spec=grid_spec,
        out_shape=[
            jax.ShapeDtypeStruct((C, T, H, W), frames.dtype),
            jax.ShapeDtypeStruct((C, num, H, W), frames.dtype),
        ],
    )(idx, frames)
    return (slow, fast)


# speedup vs baseline: 46.4308x; 1.0169x over previous
import jax
import jax.numpy as jnp
from jax.experimental import pallas as pl
from jax.experimental.pallas import tpu as pltpu

_NBUF = 16
_LAG = 8


def _pack_body(sel_ref, slot_ref, in_ref, fast_ref, slow_ref, buf, in_sems,
               out_sems, slow_sems):
    T = in_ref.shape[1]
    in_descs = [None] * T
    out_descs = [None] * T
    slow_descs = [None] * T

    def slow_start(t, b):
        slow_descs[t] = pltpu.make_async_copy(
            buf.at[b], slow_ref.at[:, pl.ds(slot_ref[t], 1)], slow_sems.at[b]
        )

        @pl.when(sel_ref[t] != 0)
        def _():
            slow_descs[t].start()

    def slow_wait(t):
        @pl.when(sel_ref[t] != 0)
        def _():
            slow_descs[t].wait()

    for i in range(T + _LAG):
        if i < T:
            b = i % _NBUF
            if i >= _NBUF:
                out_descs[i - _NBUF].wait()
                slow_wait(i - _NBUF)
            in_descs[i] = pltpu.make_async_copy(
                in_ref.at[:, pl.ds(i, 1)], buf.at[b], in_sems.at[b]
            )
            in_descs[i].start()
        j = i - _LAG
        if 0 <= j < T:
            bj = j % _NBUF
            in_descs[j].wait()
            out_descs[j] = pltpu.make_async_copy(
                buf.at[bj], fast_ref.at[:, pl.ds(j, 1)], out_sems.at[bj]
            )
            out_descs[j].start()
            slow_start(j, bj)
    for j in range(T - _NBUF, T):
        out_descs[j].wait()
        slow_wait(j)


def kernel(frames, slowfast_alpha):
    del slowfast_alpha
    C, T, H, W = frames.shape
    num = T // 4
    idx = jnp.linspace(0.0, T - 1, num).astype(jnp.int32)
    t_range = jnp.arange(T, dtype=jnp.int32)
    slot = jnp.searchsorted(idx, t_range, side="right").astype(jnp.int32) - 1
    slot = jnp.clip(slot, 0, num - 1)
    sel = (jnp.take(idx, slot) == t_range).astype(jnp.int32)

    grid_spec = pltpu.PrefetchScalarGridSpec(
        num_scalar_prefetch=2,
        grid=(1,),
        in_specs=[pl.BlockSpec(memory_space=pltpu.MemorySpace.HBM)],
        out_specs=[
            pl.BlockSpec(memory_space=pltpu.MemorySpace.HBM),
            pl.BlockSpec(memory_space=pltpu.MemorySpace.HBM),
        ],
        scratch_shapes=[
            pltpu.VMEM((_NBUF, C, 1, H, W), frames.dtype),
            pltpu.SemaphoreType.DMA((_NBUF,)),
            pltpu.SemaphoreType.DMA((_NBUF,)),
            pltpu.SemaphoreType.DMA((_NBUF,)),
        ],
    )
    fast, slow = pl.pallas_call(
        _pack_body,
        grid_spec=grid_spec,
        out_shape=[
            jax.ShapeDtypeStruct((C, T, H, W), frames.dtype),
            jax.ShapeDtypeStruct((C, num, H, W), frames.dtype),
        ],
    )(sel, slot, frames)
    return (slow, fast)
